# Initial kernel scaffold; baseline (speedup 1.0000x reference)
import jax
import jax.numpy as jnp
from jax.experimental import pallas as pl


def kernel(inputs, weight):
    raise NotImplementedError("write your pallas kernel here")


# speedup vs baseline: 7.3085x; 7.3085x over previous
import functools

import jax
import jax.numpy as jnp
from jax import lax
from jax.experimental import pallas as pl
from jax.experimental.pallas import tpu as pltpu
from jax.experimental.pallas import tpu_sc as plsc

_B, _S, _D = 4, 2048, 1024
_N = _B * _S
_NC, _NS = 2, 16
_NW = _NC * _NS
_PER_W = _N // _NW
_CHUNK = 32
_NCHUNK = _PER_W // _CHUNK

_mesh = plsc.VectorSubcoreMesh(core_axis_name="c", subcore_axis_name="s")


@functools.partial(
    pl.kernel,
    mesh=_mesh,
    out_type=jax.ShapeDtypeStruct((_N, _D), jnp.float32),
    scratch_types=[
        pltpu.VMEM((_NCHUNK, _CHUNK), jnp.int32),
        pltpu.VMEM((_CHUNK, _D), jnp.float32),
        pltpu.VMEM((_CHUNK, _D), jnp.float32),
        pltpu.SemaphoreType.DMA,
        pltpu.SemaphoreType.DMA,
        pltpu.SemaphoreType.DMA,
        pltpu.SemaphoreType.DMA,
    ],
)
def _emb_gather(idx_hbm, table_hbm, out_hbm, idx_v, buf0, buf1, g0, g1, w0, w1):
    wid = lax.axis_index("s") * _NC + lax.axis_index("c")
    base = wid * _PER_W

    pltpu.sync_copy(idx_hbm.at[wid], idx_v)

    bufs = (buf0, buf1)
    gsems = (g0, g1)
    wsems = (w0, w1)

    gh = {}
    wh = {}
    gh[0] = pltpu.async_copy(table_hbm.at[idx_v.at[0]], buf0, g0)
    gh[1] = pltpu.async_copy(table_hbm.at[idx_v.at[1]], buf1, g1)
    for j in range(_NCHUNK):
        b = j % 2
        gh[j].wait()
        wh[j] = pltpu.async_copy(
            bufs[b], out_hbm.at[pl.ds(base + j * _CHUNK, _CHUNK)], wsems[b]
        )
        if j + 2 < _NCHUNK:
            wh[j].wait()
            gh[j + 2] = pltpu.async_copy(
                table_hbm.at[idx_v.at[j + 2]], bufs[b], gsems[b]
            )
    wh[_NCHUNK - 2].wait()
    wh[_NCHUNK - 1].wait()


def kernel(inputs, weight):
    idx = inputs.astype(jnp.int32).reshape(_NW, _NCHUNK, _CHUNK)
    out = _emb_gather(idx, weight)
    return out.reshape(_B, _S, _D)


# speedup vs baseline: 7.5099x; 1.0276x over previous
import functools

import jax
import jax.numpy as jnp
from jax import lax
from jax.experimental import pallas as pl
from jax.experimental.pallas import tpu as pltpu
from jax.experimental.pallas import tpu_sc as plsc

_B, _S, _D = 4, 2048, 1024
_N = _B * _S
_NC, _NS = 2, 16
_NW = _NC * _NS
_PER_W = _N // _NW
_CHUNK = 32
_NCHUNK = _PER_W // _CHUNK
_NBUF = 3

_mesh = plsc.VectorSubcoreMesh(core_axis_name="c", subcore_axis_name="s")


@functools.partial(
    pl.kernel,
    mesh=_mesh,
    out_type=jax.ShapeDtypeStruct((_N, _D), jnp.float32),
    scratch_types=(
        [pltpu.VMEM((_NCHUNK, _CHUNK), jnp.int32)]
        + [pltpu.VMEM((_CHUNK, _D), jnp.float32) for _ in range(_NBUF)]
        + [pltpu.SemaphoreType.DMA for _ in range(2 * _NBUF)]
    ),
)
def _emb_gather(idx_hbm, table_hbm, out_hbm, idx_v, *bufs_and_sems):
    bufs = bufs_and_sems[:_NBUF]
    gsems = bufs_and_sems[_NBUF : 2 * _NBUF]
    wsems = bufs_and_sems[2 * _NBUF :]

    wid = lax.axis_index("s") * _NC + lax.axis_index("c")
    base = wid * _PER_W

    pltpu.sync_copy(idx_hbm.at[wid], idx_v)

    gh = {}
    wh = {}
    for j in range(min(_NBUF, _NCHUNK)):
        gh[j] = pltpu.async_copy(table_hbm.at[idx_v.at[j]], bufs[j], gsems[j])
    for j in range(_NCHUNK):
        b = j % _NBUF
        gh[j].wait()
        wh[j] = pltpu.async_copy(
            bufs[b], out_hbm.at[pl.ds(base + j * _CHUNK, _CHUNK)], wsems[b]
        )
        nj = j + _NBUF
        if nj < _NCHUNK:
            wh[j].wait()
            gh[nj] = pltpu.async_copy(table_hbm.at[idx_v.at[nj]], bufs[b], gsems[b])
    for j in range(max(0, _NCHUNK - _NBUF), _NCHUNK):
        wh[j].wait()


def kernel(inputs, weight):
    idx = inputs.astype(jnp.int32).reshape(_NW, _NCHUNK, _CHUNK)
    out = _emb_gather(idx, weight)
    return out.reshape(_B, _S, _D)


# speedup vs baseline: 7.5393x; 1.0039x over previous
import functools

import jax
import jax.numpy as jnp
from jax import lax
from jax.experimental import pallas as pl
from jax.experimental.pallas import tpu as pltpu
from jax.experimental.pallas import tpu_sc as plsc

_B, _S, _D = 4, 2048, 1024
_N = _B * _S
_NC, _NS = 2, 16
_NW = _NC * _NS
_PER_W = _N // _NW
_CHUNK = 16
_NCHUNK = _PER_W // _CHUNK
_NBUF = 6

_mesh = plsc.VectorSubcoreMesh(core_axis_name="c", subcore_axis_name="s")


@functools.partial(
    pl.kernel,
    mesh=_mesh,
    out_type=jax.ShapeDtypeStruct((_N, _D), jnp.float32),
    scratch_types=(
        [pltpu.VMEM((_NCHUNK, _CHUNK), jnp.int32)]
        + [pltpu.VMEM((_CHUNK, _D), jnp.float32) for _ in range(_NBUF)]
        + [pltpu.SemaphoreType.DMA for _ in range(2 * _NBUF)]
    ),
)
def _emb_gather(idx_hbm, table_hbm, out_hbm, idx_v, *bufs_and_sems):
    bufs = bufs_and_sems[:_NBUF]
    gsems = bufs_and_sems[_NBUF : 2 * _NBUF]
    wsems = bufs_and_sems[2 * _NBUF :]

    wid = lax.axis_index("s") * _NC + lax.axis_index("c")
    base = wid * _PER_W

    pltpu.sync_copy(idx_hbm.at[wid], idx_v)

    gh = {}
    wh = {}
    for j in range(min(_NBUF, _NCHUNK)):
        gh[j] = pltpu.async_copy(table_hbm.at[idx_v.at[j]], bufs[j], gsems[j])
    for j in range(_NCHUNK):
        b = j % _NBUF
        gh[j].wait()
        wh[j] = pltpu.async_copy(
            bufs[b], out_hbm.at[pl.ds(base + j * _CHUNK, _CHUNK)], wsems[b]
        )
        nj = j + _NBUF
        if nj < _NCHUNK:
            wh[j].wait()
            gh[nj] = pltpu.async_copy(table_hbm.at[idx_v.at[nj]], bufs[b], gsems[b])
    for j in range(max(0, _NCHUNK - _NBUF), _NCHUNK):
        wh[j].wait()


def kernel(inputs, weight):
    idx = inputs.astype(jnp.int32).reshape(_NW, _NCHUNK, _CHUNK)
    out = _emb_gather(idx, weight)
    return out.reshape(_B, _S, _D)


# speedup vs baseline: 7.5863x; 1.0062x over previous
import functools

import jax
import jax.numpy as jnp
from jax import lax
from jax.experimental import pallas as pl
from jax.experimental.pallas import tpu as pltpu
from jax.experimental.pallas import tpu_sc as plsc

_B, _S, _D = 4, 2048, 1024
_N = _B * _S
_NC, _NS = 2, 16
_NW = _NC * _NS
_PER_W = _N // _NW
_CHUNK = 16
_NCHUNK = _PER_W // _CHUNK
_NBUF = 6

_mesh = plsc.VectorSubcoreMesh(core_axis_name="c", subcore_axis_name="s")


@functools.partial(
    pl.kernel,
    mesh=_mesh,
    out_type=jax.ShapeDtypeStruct((_N, _D), jnp.float32),
    scratch_types=(
        [pltpu.VMEM((_PER_W,), jnp.int32)]
        + [pltpu.VMEM((_CHUNK, _D), jnp.float32) for _ in range(_NBUF)]
        + [pltpu.SemaphoreType.DMA for _ in range(2 * _NBUF)]
    ),
)
def _emb_gather(idx_hbm, table_hbm, out_hbm, idx_v, *bufs_and_sems):
    bufs = bufs_and_sems[:_NBUF]
    gsems = bufs_and_sems[_NBUF : 2 * _NBUF]
    wsems = bufs_and_sems[2 * _NBUF :]

    wid = lax.axis_index("s") * _NC + lax.axis_index("c")
    base = wid * _PER_W

    pltpu.sync_copy(idx_hbm.at[pl.ds(base, _PER_W)], idx_v)

    gh = {}
    wh = {}
    for j in range(min(_NBUF, _NCHUNK)):
        gh[j] = pltpu.async_copy(
            table_hbm.at[idx_v.at[pl.ds(j * _CHUNK, _CHUNK)]], bufs[j], gsems[j]
        )
    for j in range(_NCHUNK):
        b = j % _NBUF
        gh[j].wait()
        wh[j] = pltpu.async_copy(
            bufs[b], out_hbm.at[pl.ds(base + j * _CHUNK, _CHUNK)], wsems[b]
        )
        nj = j + _NBUF
        if nj < _NCHUNK:
            wh[j].wait()
            gh[nj] = pltpu.async_copy(
                table_hbm.at[idx_v.at[pl.ds(nj * _CHUNK, _CHUNK)]], bufs[b], gsems[b]
            )
    for j in range(max(0, _NCHUNK - _NBUF), _NCHUNK):
        wh[j].wait()


def kernel(inputs, weight):
    idx = inputs.astype(jnp.int32).reshape(-1)
    out = _emb_gather(idx, weight)
    return out.reshape(_B, _S, _D)
